# baseline (speedup 1.0000x reference)
import functools

import jax
import jax.numpy as jnp
from jax import lax
from jax.experimental import pallas as pl
from jax.experimental.pallas import tpu as pltpu
from jax.experimental.pallas import tpu_sc as plsc

B = 4096
L = 200
N = B * L
EMB = 64
PD = 32
OUT_D = EMB + 2 * PD
NPOS = 400
NW = 32
G = 128
NG = N // (NW * G)
NSEG = 4
SEGG = NG // NSEG
NBUF = 5
AHEAD = 3
NCHUNK = G // 16

_mesh = plsc.VectorSubcoreMesh(core_axis_name="c", subcore_axis_name="s")


@functools.partial(
    pl.kernel,
    mesh=_mesh,
    compiler_params=pltpu.CompilerParams(use_tc_tiling_on_sc=False,
                                         needs_layout_passes=False),
    out_type=jax.ShapeDtypeStruct((N, OUT_D), jnp.float32),
    scratch_types=[
        pltpu.VMEM((SEGG, G), jnp.int32),
        pltpu.VMEM((SEGG, G), jnp.int32),
        pltpu.VMEM((SEGG, G), jnp.int32),
        pltpu.VMEM((NPOS, PD), jnp.float32),
        pltpu.VMEM((NPOS, PD), jnp.float32),
        pltpu.VMEM((NBUF, G, EMB), jnp.float32),
        pltpu.VMEM((NBUF, G, 2 * PD), jnp.float32),
        pltpu.SemaphoreType.DMA,
        pltpu.SemaphoreType.DMA,
        pltpu.SemaphoreType.DMA,
        pltpu.SemaphoreType.DMA,
        pltpu.SemaphoreType.DMA,
        pltpu.SemaphoreType.DMA,
        pltpu.SemaphoreType.DMA,
        pltpu.SemaphoreType.DMA,
        pltpu.SemaphoreType.DMA,
        pltpu.SemaphoreType.DMA,
    ],
)
def _sc_embed(word_hbm, pos1_hbm, pos2_hbm, wtab_hbm, p1tab_hbm, p2tab_hbm,
              out_hbm, widx_v, p1idx_v, p2idx_v, p1tab_v, p2tab_v,
              wrows_v, prows_v,
              gs0, gs1, gs2, gs3, gs4, ws0, ws1, ws2, ws3, ws4):
    gsems = [gs0, gs1, gs2, gs3, gs4]
    wsems = [ws0, ws1, ws2, ws3, ws4]
    wid = lax.axis_index("s") * 2 + lax.axis_index("c")
    gbase = wid * NG

    pltpu.sync_copy(p1tab_hbm, p1tab_v)
    pltpu.sync_copy(p2tab_hbm, p2tab_v)

    lane = lax.iota(jnp.int32, 16)

    def gather_start(g, b):
        pltpu.async_copy(wtab_hbm.at[widx_v.at[g]], wrows_v.at[b], gsems[b])

    def gather_wait(b):
        pltpu.make_async_copy(wtab_hbm.at[pl.ds(0, G)], wrows_v.at[b],
                              gsems[b]).wait()

    def pos_compute(g, b):
        pbuf = prows_v.at[b]

        def chunk(c, carry):
            rows = lane + c * 16
            r1 = p1idx_v[g, pl.ds(c * 16, 16)]
            r2 = p2idx_v[g, pl.ds(c * 16, 16)]
            zero16 = lane * 0
            for j in range(PD):
                colv = zero16 + j
                v1 = plsc.load_gather(p1tab_v, [r1, colv])
                plsc.store_scatter(pbuf, [rows, colv], v1)
                v2 = plsc.load_gather(p2tab_v, [r2, colv])
                plsc.store_scatter(pbuf, [rows, colv + PD], v2)
            return carry

        lax.fori_loop(0, NCHUNK, chunk, 0)

    def write_start(g, b, seg_off):
        row0 = (gbase + seg_off + g) * G
        pltpu.async_copy(wrows_v.at[b],
                         out_hbm.at[pl.ds(row0, G), pl.ds(0, EMB)], wsems[b])
        pltpu.async_copy(prows_v.at[b],
                         out_hbm.at[pl.ds(row0, G), pl.ds(EMB, 2 * PD)],
                         wsems[b])

    def write_wait(b):
        row0 = gbase * G
        pltpu.make_async_copy(wrows_v.at[b],
                              out_hbm.at[pl.ds(row0, G), pl.ds(0, EMB)],
                              wsems[b]).wait()
        pltpu.make_async_copy(prows_v.at[b],
                              out_hbm.at[pl.ds(row0, G), pl.ds(EMB, 2 * PD)],
                              wsems[b]).wait()

    for seg in range(NSEG):
        seg_off = seg * SEGG
        pltpu.sync_copy(word_hbm.at[pl.ds(gbase + seg_off, SEGG)], widx_v)
        pltpu.sync_copy(pos1_hbm.at[pl.ds(gbase + seg_off, SEGG)], p1idx_v)
        pltpu.sync_copy(pos2_hbm.at[pl.ds(gbase + seg_off, SEGG)], p2idx_v)

        for p in range(AHEAD):
            gather_start(p, p)

        def outer(i, carry):
            go = i * NBUF
            for b in range(NBUF):
                g = go + b
                pos_compute(g, b)
                gather_wait(b)
                write_start(g, b, seg_off)
                bn = (b + AHEAD) % NBUF

                @pl.when(g >= NBUF - AHEAD)
                def _():
                    write_wait(bn)

                @pl.when(g + AHEAD < SEGG)
                def _():
                    gather_start(g + AHEAD, bn)
            return carry

        lax.fori_loop(0, SEGG // NBUF, outer, 0)
        write_wait((SEGG - 2) % NBUF)
        write_wait((SEGG - 1) % NBUF)


VOCAB = 1000000


def kernel(word, pos1, pos2, word_table, pos1_table, pos2_table):
    word2d = jnp.reshape(word, (N // G, G))
    pos1_2d = jnp.reshape(pos1, (N // G, G))
    pos2_2d = jnp.reshape(pos2, (N // G, G))
    out = _sc_embed(word2d, pos1_2d, pos2_2d,
                    word_table, pos1_table, pos2_table)
    return jnp.reshape(out, (B, L, OUT_D))


# speedup vs baseline: 1.9939x; 1.9939x over previous
import functools

import jax
import jax.numpy as jnp
from jax import lax
from jax.experimental import pallas as pl
from jax.experimental.pallas import tpu as pltpu
from jax.experimental.pallas import tpu_sc as plsc

B = 4096
L = 200
N = B * L
EMB = 64
PD = 32
OUT_D = EMB + 2 * PD
NW = 32
G = 128
NG = N // (NW * G)
NSEG = 2
SEGG = NG // NSEG
NBUF = 5
AHEAD = 3

_mesh = plsc.VectorSubcoreMesh(core_axis_name="c", subcore_axis_name="s")


@functools.partial(
    pl.kernel,
    mesh=_mesh,
    compiler_params=pltpu.CompilerParams(use_tc_tiling_on_sc=False),
    out_type=jax.ShapeDtypeStruct((N, OUT_D), jnp.float32),
    scratch_types=[
        pltpu.VMEM((SEGG, G), jnp.int32),
        pltpu.VMEM((SEGG, G), jnp.int32),
        pltpu.VMEM((SEGG, G), jnp.int32),
        pltpu.VMEM((NBUF, G, EMB), jnp.float32),
        pltpu.VMEM((NBUF, G, PD), jnp.float32),
        pltpu.VMEM((NBUF, G, PD), jnp.float32),
        pltpu.SemaphoreType.DMA,
        pltpu.SemaphoreType.DMA,
        pltpu.SemaphoreType.DMA,
        pltpu.SemaphoreType.DMA,
        pltpu.SemaphoreType.DMA,
        pltpu.SemaphoreType.DMA,
        pltpu.SemaphoreType.DMA,
        pltpu.SemaphoreType.DMA,
        pltpu.SemaphoreType.DMA,
        pltpu.SemaphoreType.DMA,
    ],
)
def _sc_embed(word_hbm, pos1_hbm, pos2_hbm, wtab_hbm, p1tab_hbm, p2tab_hbm,
              out_hbm, widx_v, p1idx_v, p2idx_v, wrows_v, p1rows_v, p2rows_v,
              gs0, gs1, gs2, gs3, gs4, ws0, ws1, ws2, ws3, ws4):
    gsems = [gs0, gs1, gs2, gs3, gs4]
    wsems = [ws0, ws1, ws2, ws3, ws4]
    wid = lax.axis_index("s") * 2 + lax.axis_index("c")
    gbase = wid * NG

    def gather_start(g, b):
        pltpu.async_copy(wtab_hbm.at[widx_v.at[g]], wrows_v.at[b], gsems[b])
        pltpu.async_copy(p1tab_hbm.at[p1idx_v.at[g]], p1rows_v.at[b], gsems[b])
        pltpu.async_copy(p2tab_hbm.at[p2idx_v.at[g]], p2rows_v.at[b], gsems[b])

    def gather_wait(b):
        pltpu.make_async_copy(wtab_hbm.at[pl.ds(0, G)], wrows_v.at[b],
                              gsems[b]).wait()
        pltpu.make_async_copy(p1tab_hbm.at[pl.ds(0, G)], p1rows_v.at[b],
                              gsems[b]).wait()
        pltpu.make_async_copy(p2tab_hbm.at[pl.ds(0, G)], p2rows_v.at[b],
                              gsems[b]).wait()

    def write_start(g, b, seg_off):
        row0 = (gbase + seg_off + g) * G
        pltpu.async_copy(wrows_v.at[b],
                         out_hbm.at[pl.ds(row0, G), pl.ds(0, EMB)], wsems[b])
        pltpu.async_copy(p1rows_v.at[b],
                         out_hbm.at[pl.ds(row0, G), pl.ds(EMB, PD)], wsems[b])
        pltpu.async_copy(p2rows_v.at[b],
                         out_hbm.at[pl.ds(row0, G), pl.ds(EMB + PD, PD)],
                         wsems[b])

    def write_wait(b):
        row0 = gbase * G
        pltpu.make_async_copy(wrows_v.at[b],
                              out_hbm.at[pl.ds(row0, G), pl.ds(0, EMB)],
                              wsems[b]).wait()
        pltpu.make_async_copy(p1rows_v.at[b],
                              out_hbm.at[pl.ds(row0, G), pl.ds(EMB, PD)],
                              wsems[b]).wait()
        pltpu.make_async_copy(p2rows_v.at[b],
                              out_hbm.at[pl.ds(row0, G), pl.ds(EMB + PD, PD)],
                              wsems[b]).wait()

    for seg in range(NSEG):
        seg_off = seg * SEGG
        pltpu.sync_copy(word_hbm.at[pl.ds(gbase + seg_off, SEGG)], widx_v)
        pltpu.sync_copy(pos1_hbm.at[pl.ds(gbase + seg_off, SEGG)], p1idx_v)
        pltpu.sync_copy(pos2_hbm.at[pl.ds(gbase + seg_off, SEGG)], p2idx_v)

        for p in range(AHEAD):
            gather_start(p, p)

        def outer(i, carry):
            go = i * NBUF
            for b in range(NBUF):
                g = go + b
                gather_wait(b)
                write_start(g, b, seg_off)
                bn = (b + AHEAD) % NBUF

                @pl.when(g >= NBUF - AHEAD)
                def _():
                    write_wait(bn)

                @pl.when(g + AHEAD < SEGG)
                def _():
                    gather_start(g + AHEAD, bn)
            return carry

        lax.fori_loop(0, SEGG // NBUF, outer, 0)
        write_wait((SEGG - 2) % NBUF)
        write_wait((SEGG - 1) % NBUF)


def kernel(word, pos1, pos2, word_table, pos1_table, pos2_table):
    word2d = jnp.reshape(word, (N // G, G))
    pos1_2d = jnp.reshape(pos1, (N // G, G))
    pos2_2d = jnp.reshape(pos2, (N // G, G))
    out = _sc_embed(word2d, pos1_2d, pos2_2d,
                    word_table, pos1_table, pos2_table)
    return jnp.reshape(out, (B, L, OUT_D))
